# Initial kernel scaffold; baseline (speedup 1.0000x reference)
import jax
import jax.numpy as jnp
from jax.experimental import pallas as pl


def kernel(x, edge_index_rel0, edge_index_rel1, params):
    raise NotImplementedError("write your pallas kernel here")


# speedup vs baseline: 60.8474x; 60.8474x over previous
import functools

import jax
import jax.numpy as jnp
from jax import lax
from jax.experimental import pallas as pl
from jax.experimental.pallas import tpu as pltpu
from jax.experimental.pallas import tpu_sc as plsc

N = 10000
E = 160000
D = 128
H = 16
DH = 8
NLAYERS = 4
HID2 = 64
OUT = 64

NTILES = 16
EPT = E // NTILES
CHUNK = 80
NCH = EPT // CHUNK
ROWS_PT = 624
ROWS_REM = N - ROWS_PT * NTILES


def _bn(h, g, b):
    mu = jnp.mean(h, axis=0)
    var = jnp.mean((h - mu) * (h - mu), axis=0)
    return (h - mu) / jnp.sqrt(var + 1e-5) * g + b


def _embed_body(x_ref, w1, b1, g, b, w2, b2, o_ref):
    h = jnp.dot(x_ref[...], w1[...], preferred_element_type=jnp.float32) + b1[...]
    h = _bn(h, g[...], b[...])
    h = jnp.maximum(h, 0.0)
    o_ref[...] = (
        jnp.dot(h, w2[...], preferred_element_type=jnp.float32) + b2[...] + h
    )


_embed_call = pl.pallas_call(
    _embed_body,
    out_shape=jax.ShapeDtypeStruct((N, D), jnp.float32),
)


def _layer_a_body(h_ref, w0, w1, al0, ar0, al1, ar1,
                  z0_ref, z1_ref, el0_ref, er0_ref, el1_ref, er1_ref,
                  m0_ref, m1_ref):
    h = h_ref[...]
    z0_ref[...] = jnp.dot(h, w0[...], preferred_element_type=jnp.float32)
    z1_ref[...] = jnp.dot(h, w1[...], preferred_element_type=jnp.float32)
    el0 = jnp.dot(h, al0[...], preferred_element_type=jnp.float32)
    er0 = jnp.dot(h, ar0[...], preferred_element_type=jnp.float32)
    el1 = jnp.dot(h, al1[...], preferred_element_type=jnp.float32)
    er1 = jnp.dot(h, ar1[...], preferred_element_type=jnp.float32)
    el0_ref[...] = el0
    er0_ref[...] = er0
    el1_ref[...] = el1
    er1_ref[...] = er1
    q0 = jnp.max(el0, axis=0) + jnp.max(er0, axis=0)
    q1 = jnp.max(el1, axis=0) + jnp.max(er1, axis=0)
    m0_ref[...] = jnp.maximum(q0, 0.2 * q0)
    m1_ref[...] = jnp.maximum(q1, 0.2 * q1)


_layer_a_call = pl.pallas_call(
    _layer_a_body,
    out_shape=[
        jax.ShapeDtypeStruct((N, D), jnp.float32),
        jax.ShapeDtypeStruct((N, D), jnp.float32),
        jax.ShapeDtypeStruct((N, H), jnp.float32),
        jax.ShapeDtypeStruct((N, H), jnp.float32),
        jax.ShapeDtypeStruct((N, H), jnp.float32),
        jax.ShapeDtypeStruct((N, H), jnp.float32),
        jax.ShapeDtypeStruct((H,), jnp.float32),
        jax.ShapeDtypeStruct((H,), jnp.float32),
    ],
)


_BROWS = 1000


def _layer_b1_body(a0, a1, s0, s1, bias0, bias1, o_ref):
    s0t = jnp.concatenate([s0[...]] * DH, axis=1)
    s1t = jnp.concatenate([s1[...]] * DH, axis=1)
    rel = (a0[...] / s0t + bias0[...]) + (a1[...] / s1t + bias1[...])
    o_ref[...] = jnp.maximum(rel, 0.01 * rel)


_layer_b1_call = pl.pallas_call(
    _layer_b1_body,
    grid=(N // _BROWS,),
    in_specs=[
        pl.BlockSpec((_BROWS, D), lambda i: (i, 0)),
        pl.BlockSpec((_BROWS, D), lambda i: (i, 0)),
        pl.BlockSpec((_BROWS, H), lambda i: (i, 0)),
        pl.BlockSpec((_BROWS, H), lambda i: (i, 0)),
        pl.BlockSpec((D,), lambda i: (0,)),
        pl.BlockSpec((D,), lambda i: (0,)),
    ],
    out_specs=pl.BlockSpec((_BROWS, D), lambda i: (i, 0)),
    out_shape=jax.ShapeDtypeStruct((N, D), jnp.float32),
)


def _layer_b2_body(v_ref, h_ref, w1, b1, g, b, w2, b2, o_ref):
    hh = jnp.dot(v_ref[...], w1[...], preferred_element_type=jnp.float32) + b1[...]
    hh = _bn(hh, g[...], b[...])
    hh = jnp.maximum(hh, 0.0)
    o_ref[...] = (
        jnp.dot(hh, w2[...], preferred_element_type=jnp.float32)
        + b2[...]
        + h_ref[...]
    )


_layer_b2_call = pl.pallas_call(
    _layer_b2_body,
    out_shape=jax.ShapeDtypeStruct((N, D), jnp.float32),
)


def _dec_body(h_ref, w1, b1, g, b, w2, b2, o_ref):
    hh = jnp.dot(h_ref[...], w1[...], preferred_element_type=jnp.float32) + b1[...]
    hh = _bn(hh, g[...], b[...])
    hh = jnp.maximum(hh, 0.0)
    o_ref[...] = jnp.dot(hh, w2[...], preferred_element_type=jnp.float32) + b2[...]


_dec_call = pl.pallas_call(
    _dec_body,
    out_shape=jax.ShapeDtypeStruct((N, OUT), jnp.float32),
)


_sc_mesh = plsc.VectorSubcoreMesh(
    core_axis_name="c", subcore_axis_name="s", num_cores=2, num_subcores=16
)


def _edge_body(z0, el0, er0, src0, dst0, m0, z1, el1, er1, src1, dst1, m1,
               zf, zs,
               acc0_out, acc1_out, s0_out, s1_out,
               acc_sh, s_sh, idxs, idxd, elb, erb, eeb, zb, mb,
               sem_el, sem_er, sem_z):
    cid = lax.axis_index("c")
    sid = lax.axis_index("s")
    rows = pl.ds(sid * ROWS_PT, ROWS_PT)
    tail = pl.ds(ROWS_PT * NTILES, ROWS_REM)

    def do_rel(z_h, el_h, er_h, src_h, dst_h, m_h, acc_out, s_out):
        pltpu.sync_copy(zf.at[rows], acc_sh.at[rows])
        pltpu.sync_copy(zs.at[rows], s_sh.at[rows])

        @pl.when(sid == NTILES - 1)
        def _():
            pltpu.sync_copy(zf.at[tail], acc_sh.at[tail])
            pltpu.sync_copy(zs.at[tail], s_sh.at[tail])

        pltpu.sync_copy(m_h, mb)
        plsc.subcore_barrier()
        mv = mb[...]

        def chunk(i, carry):
            base = sid * EPT + i * CHUNK
            pltpu.sync_copy(src_h.at[pl.ds(base, CHUNK)], idxs)
            pltpu.sync_copy(dst_h.at[pl.ds(base, CHUNK)], idxd)
            cp_el = pltpu.async_copy(el_h.at[idxs], elb, sem_el)
            cp_er = pltpu.async_copy(er_h.at[idxd], erb, sem_er)
            cp_z = pltpu.async_copy(z_h.at[idxs], zb, sem_z)
            cp_el.wait()
            cp_er.wait()
            cp_z.wait()

            def edge(j, c):
                ev = elb[j, :] + erb[j, :]
                ev = jnp.maximum(ev, 0.2 * ev)
                ee = jnp.exp(ev - mv)
                eeb[j, :] = ee
                for k in range(DH):
                    sl = pl.ds(k * 16, 16)
                    zb[j, sl] = zb[j, sl] * ee
                return c

            lax.fori_loop(0, CHUNK, edge, 0)
            pltpu.sync_copy(eeb, s_sh.at[idxd], add=True)
            pltpu.sync_copy(zb, acc_sh.at[idxd], add=True)
            return carry

        lax.fori_loop(0, NCH, chunk, 0)
        plsc.subcore_barrier()
        pltpu.sync_copy(acc_sh.at[rows], acc_out.at[rows])
        pltpu.sync_copy(s_sh.at[rows], s_out.at[rows])

        @pl.when(sid == NTILES - 1)
        def _():
            pltpu.sync_copy(acc_sh.at[tail], acc_out.at[tail])
            pltpu.sync_copy(s_sh.at[tail], s_out.at[tail])

    @pl.when(cid == 0)
    def _():
        do_rel(z0, el0, er0, src0, dst0, m0, acc0_out, s0_out)

    @pl.when(cid == 1)
    def _():
        do_rel(z1, el1, er1, src1, dst1, m1, acc1_out, s1_out)


_edge_call = pl.kernel(
    _edge_body,
    out_type=[
        jax.ShapeDtypeStruct((N, D), jnp.float32),
        jax.ShapeDtypeStruct((N, D), jnp.float32),
        jax.ShapeDtypeStruct((N, H), jnp.float32),
        jax.ShapeDtypeStruct((N, H), jnp.float32),
    ],
    mesh=_sc_mesh,
    compiler_params=pltpu.CompilerParams(use_tc_tiling_on_sc=False),
    scratch_types=[
        pltpu.VMEM_SHARED((N, D), jnp.float32),
        pltpu.VMEM_SHARED((N, H), jnp.float32),
        pltpu.VMEM((CHUNK,), jnp.int32),
        pltpu.VMEM((CHUNK,), jnp.int32),
        pltpu.VMEM((CHUNK, H), jnp.float32),
        pltpu.VMEM((CHUNK, H), jnp.float32),
        pltpu.VMEM((CHUNK, H), jnp.float32),
        pltpu.VMEM((CHUNK, D), jnp.float32),
        pltpu.VMEM((H,), jnp.float32),
        pltpu.SemaphoreType.DMA,
        pltpu.SemaphoreType.DMA,
        pltpu.SemaphoreType.DMA,
    ],
)


def kernel(x, edge_index_rel0, edge_index_rel1, params):
    perm = jnp.array([(f % H) * DH + f // H for f in range(D)], dtype=jnp.int32)

    src0 = edge_index_rel0[0]
    dst0 = edge_index_rel0[1]
    src1 = edge_index_rel1[0]
    dst1 = edge_index_rel1[1]
    zf = jnp.zeros((N, D), jnp.float32)
    zs = jnp.zeros((N, H), jnp.float32)

    pe = params["embed"]
    h = _embed_call(x, pe["fc1_w"], pe["fc1_b"], pe["bn_g"], pe["bn_b"],
                    pe["fc2_w"], pe["fc2_b"])

    for l in range(NLAYERS):
        g0 = params["gat"][l][0]
        g1 = params["gat"][l][1]
        w0p = g0["w"][:, perm]
        w1p = g1["w"][:, perm]
        a_l0 = jnp.einsum("dhk,hk->dh", g0["w"].reshape(D, H, DH), g0["al"])
        a_r0 = jnp.einsum("dhk,hk->dh", g0["w"].reshape(D, H, DH), g0["ar"])
        a_l1 = jnp.einsum("dhk,hk->dh", g1["w"].reshape(D, H, DH), g1["al"])
        a_r1 = jnp.einsum("dhk,hk->dh", g1["w"].reshape(D, H, DH), g1["ar"])
        bias0p = g0["bias"][perm]
        bias1p = g1["bias"][perm]

        z0, z1, el0, er0, el1, er1, m0, m1 = _layer_a_call(
            h, w0p, w1p, a_l0, a_r0, a_l1, a_r1
        )
        acc0, acc1, s0, s1 = _edge_call(
            z0, el0, er0, src0, dst0, m0, z1, el1, er1, src1, dst1, m1, zf, zs
        )
        pm = params["mlp"][l]
        fc1p_w = pm["fc1_w"][perm, :]
        v = _layer_b1_call(acc0, acc1, s0, s1, bias0p, bias1p)
        h = _layer_b2_call(
            v, h, fc1p_w, pm["fc1_b"], pm["bn_g"], pm["bn_b"], pm["fc2_w"],
            pm["fc2_b"]
        )

    pd = params["dec"]
    return _dec_call(h, pd["fc1_w"], pd["fc1_b"], pd["bn_g"], pd["bn_b"],
                     pd["fc2_w"], pd["fc2_b"])


# speedup vs baseline: 145.9143x; 2.3980x over previous
import functools

import jax
import jax.numpy as jnp
from jax import lax
from jax.experimental import pallas as pl
from jax.experimental.pallas import tpu as pltpu
from jax.experimental.pallas import tpu_sc as plsc

N = 10000
E = 160000
D = 128
H = 16
DH = 8
NLAYERS = 4
HID2 = 64
OUT = 64

NTILES = 16
EPT = E // NTILES
CHUNK = 40
NCH = EPT // CHUNK
NBUF = 5
DA = D + H
ROWS_PT = 624
ROWS_REM = N - ROWS_PT * NTILES


def _bn(h, g, b):
    mu = jnp.mean(h, axis=0)
    var = jnp.mean((h - mu) * (h - mu), axis=0)
    return (h - mu) / jnp.sqrt(var + 1e-5) * g + b


def _embed_body(x_ref, w1, b1, g, b, w2, b2, o_ref):
    h = jnp.dot(x_ref[...], w1[...], preferred_element_type=jnp.float32) + b1[...]
    h = _bn(h, g[...], b[...])
    h = jnp.maximum(h, 0.0)
    o_ref[...] = (
        jnp.dot(h, w2[...], preferred_element_type=jnp.float32) + b2[...] + h
    )


_embed_call = pl.pallas_call(
    _embed_body,
    out_shape=jax.ShapeDtypeStruct((N, D), jnp.float32),
)


def _layer_a_body(h_ref, w0, w1, al0, ar0, al1, ar1,
                  za0_ref, za1_ref, er0_ref, er1_ref, m0_ref, m1_ref):
    h = h_ref[...]
    z0 = jnp.dot(h, w0[...], preferred_element_type=jnp.float32)
    z1 = jnp.dot(h, w1[...], preferred_element_type=jnp.float32)
    el0 = jnp.dot(h, al0[...], preferred_element_type=jnp.float32)
    er0 = jnp.dot(h, ar0[...], preferred_element_type=jnp.float32)
    el1 = jnp.dot(h, al1[...], preferred_element_type=jnp.float32)
    er1 = jnp.dot(h, ar1[...], preferred_element_type=jnp.float32)
    za0_ref[...] = jnp.concatenate([z0, el0], axis=1)
    za1_ref[...] = jnp.concatenate([z1, el1], axis=1)
    er0_ref[...] = er0
    er1_ref[...] = er1
    q0 = jnp.max(el0, axis=0) + jnp.max(er0, axis=0)
    q1 = jnp.max(el1, axis=0) + jnp.max(er1, axis=0)
    m0_ref[...] = jnp.maximum(q0, 0.2 * q0)
    m1_ref[...] = jnp.maximum(q1, 0.2 * q1)


_layer_a_call = pl.pallas_call(
    _layer_a_body,
    out_shape=[
        jax.ShapeDtypeStruct((N, DA), jnp.float32),
        jax.ShapeDtypeStruct((N, DA), jnp.float32),
        jax.ShapeDtypeStruct((N, H), jnp.float32),
        jax.ShapeDtypeStruct((N, H), jnp.float32),
        jax.ShapeDtypeStruct((H,), jnp.float32),
        jax.ShapeDtypeStruct((H,), jnp.float32),
    ],
)


_BROWS = 1000


def _layer_b1_body(a0, a1, bias0, bias1, o_ref):
    acc0 = a0[:, :D]
    acc1 = a1[:, :D]
    s0t = jnp.concatenate([a0[:, D:]] * DH, axis=1)
    s1t = jnp.concatenate([a1[:, D:]] * DH, axis=1)
    rel = (acc0 / s0t + bias0[...]) + (acc1 / s1t + bias1[...])
    o_ref[...] = jnp.maximum(rel, 0.01 * rel)


_layer_b1_call = pl.pallas_call(
    _layer_b1_body,
    grid=(N // _BROWS,),
    in_specs=[
        pl.BlockSpec((_BROWS, DA), lambda i: (i, 0)),
        pl.BlockSpec((_BROWS, DA), lambda i: (i, 0)),
        pl.BlockSpec((D,), lambda i: (0,)),
        pl.BlockSpec((D,), lambda i: (0,)),
    ],
    out_specs=pl.BlockSpec((_BROWS, D), lambda i: (i, 0)),
    out_shape=jax.ShapeDtypeStruct((N, D), jnp.float32),
)


def _layer_b2_body(v_ref, h_ref, w1, b1, g, b, w2, b2, o_ref):
    hh = jnp.dot(v_ref[...], w1[...], preferred_element_type=jnp.float32) + b1[...]
    hh = _bn(hh, g[...], b[...])
    hh = jnp.maximum(hh, 0.0)
    o_ref[...] = (
        jnp.dot(hh, w2[...], preferred_element_type=jnp.float32)
        + b2[...]
        + h_ref[...]
    )


_layer_b2_call = pl.pallas_call(
    _layer_b2_body,
    out_shape=jax.ShapeDtypeStruct((N, D), jnp.float32),
)


def _dec_body(h_ref, w1, b1, g, b, w2, b2, o_ref):
    hh = jnp.dot(h_ref[...], w1[...], preferred_element_type=jnp.float32) + b1[...]
    hh = _bn(hh, g[...], b[...])
    hh = jnp.maximum(hh, 0.0)
    o_ref[...] = jnp.dot(hh, w2[...], preferred_element_type=jnp.float32) + b2[...]


_dec_call = pl.pallas_call(
    _dec_body,
    out_shape=jax.ShapeDtypeStruct((N, OUT), jnp.float32),
)


_sc_mesh = plsc.VectorSubcoreMesh(
    core_axis_name="c", subcore_axis_name="s", num_cores=2, num_subcores=16
)


def _edge_body(za0, er0, src0, dst0, m0, za1, er1, src1, dst1, m1, zf,
               acc0_out, acc1_out,
               acc_sh, isrc, idst, erb, zbe, mb,
               gs0, gs1, gs2, gs3, gs4, is0, is1, is2, is3, is4):
    cid = lax.axis_index("c")
    sid = lax.axis_index("s")
    rows = pl.ds(sid * ROWS_PT, ROWS_PT)
    tail = pl.ds(ROWS_PT * NTILES, ROWS_REM)
    gsem = [gs0, gs1, gs2, gs3, gs4]
    isem = [is0, is1, is2, is3, is4]

    def do_rel(za_h, er_h, src_h, dst_h, m_h, acc_out):
        pltpu.sync_copy(zf.at[rows], acc_sh.at[rows])

        @pl.when(sid == NTILES - 1)
        def _():
            pltpu.sync_copy(zf.at[tail], acc_sh.at[tail])

        pltpu.sync_copy(m_h, mb)
        plsc.subcore_barrier()
        mv = mb[...]

        def issue_idx(i, b):
            r = sid * NCH + i
            pltpu.async_copy(src_h.at[r], isrc.at[b], isem[b])
            pltpu.async_copy(dst_h.at[r], idst.at[b], isem[b])

        def wait_idx(i, b):
            r = sid * NCH + i
            pltpu.make_async_copy(src_h.at[r], isrc.at[b], isem[b]).wait()
            pltpu.make_async_copy(dst_h.at[r], idst.at[b], isem[b]).wait()

        def issue_gather(b):
            pltpu.async_copy(za_h.at[isrc.at[b]], zbe.at[b], gsem[b])
            pltpu.async_copy(er_h.at[idst.at[b]], erb.at[b], gsem[b])

        def wait_gather(b):
            pltpu.make_async_copy(za_h.at[isrc.at[b]], zbe.at[b], gsem[b]).wait()
            pltpu.make_async_copy(er_h.at[idst.at[b]], erb.at[b], gsem[b]).wait()

        for b in range(NBUF - 1):
            issue_idx(b, b)
        for b in range(2):
            wait_idx(b, b)
            issue_gather(b)

        def outer(g, carry):
            for b in range(NBUF):
                i = g * NBUF + b
                wait_gather(b)

                @pl.when(i + NBUF - 1 < NCH)
                def _(b4=(b + NBUF - 1) % NBUF, i4=i + NBUF - 1):
                    issue_idx(i4, b4)

                @pl.when(i + 2 < NCH)
                def _(b2=(b + 2) % NBUF, i2=i + 2):
                    wait_idx(i2, b2)
                    issue_gather(b2)

                @plsc.parallel_loop(0, CHUNK, 1, unroll=2)
                def _(j, b=b):
                    sle = pl.ds(D, H)
                    ev = zbe[b, j, sle] + erb[b, j, :]
                    ev = jnp.maximum(ev, 0.2 * ev)
                    ee = jnp.exp(ev - mv)
                    zbe[b, j, sle] = ee
                    for k in range(DH):
                        sl = pl.ds(k * 16, 16)
                        zbe[b, j, sl] = zbe[b, j, sl] * ee

                pltpu.sync_copy(zbe.at[b], acc_sh.at[idst.at[b]], add=True)
            return carry

        lax.fori_loop(0, NCH // NBUF, outer, 0)
        plsc.subcore_barrier()
        pltpu.sync_copy(acc_sh.at[rows], acc_out.at[rows])

        @pl.when(sid == NTILES - 1)
        def _():
            pltpu.sync_copy(acc_sh.at[tail], acc_out.at[tail])

    @pl.when(cid == 0)
    def _():
        do_rel(za0, er0, src0, dst0, m0, acc0_out)

    @pl.when(cid == 1)
    def _():
        do_rel(za1, er1, src1, dst1, m1, acc1_out)


_edge_call = pl.kernel(
    _edge_body,
    out_type=[
        jax.ShapeDtypeStruct((N, DA), jnp.float32),
        jax.ShapeDtypeStruct((N, DA), jnp.float32),
    ],
    mesh=_sc_mesh,
    compiler_params=pltpu.CompilerParams(use_tc_tiling_on_sc=False),
    scratch_types=[
        pltpu.VMEM_SHARED((N, DA), jnp.float32),
        pltpu.VMEM((NBUF, CHUNK), jnp.int32),
        pltpu.VMEM((NBUF, CHUNK), jnp.int32),
        pltpu.VMEM((NBUF, CHUNK, H), jnp.float32),
        pltpu.VMEM((NBUF, CHUNK, DA), jnp.float32),
        pltpu.VMEM((H,), jnp.float32),
        pltpu.SemaphoreType.DMA,
        pltpu.SemaphoreType.DMA,
        pltpu.SemaphoreType.DMA,
        pltpu.SemaphoreType.DMA,
        pltpu.SemaphoreType.DMA,
        pltpu.SemaphoreType.DMA,
        pltpu.SemaphoreType.DMA,
        pltpu.SemaphoreType.DMA,
        pltpu.SemaphoreType.DMA,
        pltpu.SemaphoreType.DMA,
    ],
)


def kernel(x, edge_index_rel0, edge_index_rel1, params):
    perm = jnp.array([(f % H) * DH + f // H for f in range(D)], dtype=jnp.int32)

    src0 = edge_index_rel0[0].reshape(E // CHUNK, CHUNK)
    dst0 = edge_index_rel0[1].reshape(E // CHUNK, CHUNK)
    src1 = edge_index_rel1[0].reshape(E // CHUNK, CHUNK)
    dst1 = edge_index_rel1[1].reshape(E // CHUNK, CHUNK)
    zf = jnp.zeros((N, DA), jnp.float32)

    pe = params["embed"]
    h = _embed_call(x, pe["fc1_w"], pe["fc1_b"], pe["bn_g"], pe["bn_b"],
                    pe["fc2_w"], pe["fc2_b"])

    for l in range(NLAYERS):
        g0 = params["gat"][l][0]
        g1 = params["gat"][l][1]
        w0p = g0["w"][:, perm]
        w1p = g1["w"][:, perm]
        a_l0 = jnp.einsum("dhk,hk->dh", g0["w"].reshape(D, H, DH), g0["al"])
        a_r0 = jnp.einsum("dhk,hk->dh", g0["w"].reshape(D, H, DH), g0["ar"])
        a_l1 = jnp.einsum("dhk,hk->dh", g1["w"].reshape(D, H, DH), g1["al"])
        a_r1 = jnp.einsum("dhk,hk->dh", g1["w"].reshape(D, H, DH), g1["ar"])
        bias0p = g0["bias"][perm]
        bias1p = g1["bias"][perm]

        za0, za1, er0, er1, m0, m1 = _layer_a_call(
            h, w0p, w1p, a_l0, a_r0, a_l1, a_r1
        )
        acc0, acc1 = _edge_call(
            za0, er0, src0, dst0, m0, za1, er1, src1, dst1, m1, zf
        )
        pm = params["mlp"][l]
        fc1p_w = pm["fc1_w"][perm, :]
        v = _layer_b1_call(acc0, acc1, bias0p, bias1p)
        h = _layer_b2_call(
            v, h, fc1p_w, pm["fc1_b"], pm["bn_g"], pm["bn_b"], pm["fc2_w"],
            pm["fc2_b"]
        )

    pd = params["dec"]
    return _dec_call(h, pd["fc1_w"], pd["fc1_b"], pd["bn_g"], pd["bn_b"],
                     pd["fc2_w"], pd["fc2_b"])
